# baseline (speedup 1.0000x reference)
import functools

import jax
import jax.numpy as jnp
from jax.experimental import pallas as pl
from jax.experimental.pallas import tpu as pltpu


def _fused_kernel(x_ref, wp_ref, o_ref, *, cin, cout, ctmp, h, w, lane_tile):
    hw = h * w
    w1 = wp_ref[0:ctmp, :].astype(jnp.bfloat16)
    w2 = wp_ref[ctmp:ctmp + cout, 0:ctmp].astype(jnp.bfloat16)
    b2 = wp_ref[ctmp:ctmp + cout, ctmp:ctmp + 1].astype(jnp.bfloat16)
    alpha = wp_ref[ctmp:ctmp + 1, ctmp + 1:ctmp + 2][0, 0].astype(jnp.bfloat16)
    wc = jnp.dot(w2, w1, preferred_element_type=jnp.float32)
    wc = wc.astype(jnp.bfloat16)

    xb = x_ref[0].astype(jnp.bfloat16).reshape(cin, hw)
    zrow = jnp.zeros((cin, w), jnp.bfloat16)
    xp = jnp.concatenate([zrow, xb, zrow], axis=1)
    ln = hw + 2 * w

    col = jax.lax.broadcasted_iota(jnp.int32, (cin, ln), 1) & (w - 1)
    z1 = jnp.zeros((cin, 1), jnp.bfloat16)
    zb = jnp.bfloat16(0)
    d0 = jnp.concatenate([z1, xp[:, :-1]], axis=1)
    d0 = jnp.where(col == 0, zb, d0)
    d2 = jnp.concatenate([xp[:, 1:], z1], axis=1)
    d2 = jnp.where(col == w - 1, zb, d2)
    dstack = jnp.concatenate([d0, xp, d2], axis=0)

    for p0 in range(0, hw, lane_tile):
        y = None
        for dy in range(3):
            rhs = dstack[:, p0 + dy * w: p0 + dy * w + lane_tile]
            part = jnp.dot(wc[:, dy * 3 * cin:(dy + 1) * 3 * cin], rhs,
                           preferred_element_type=jnp.float32)
            y = part if y is None else y + part
        y = y.astype(jnp.bfloat16) + b2
        y = jnp.where(y >= 0, y, alpha * y)
        y = y.reshape(cout, lane_tile // w, w).astype(o_ref.dtype)
        o_ref[0, :, p0 // w:(p0 + lane_tile) // w, :] = y


def kernel(x_nchw, w1_torch, w2_torch, b2, alpha):
    n, cin, h, w = x_nchw.shape
    ctmp = w1_torch.shape[0]
    cout = w2_torch.shape[0]
    k = w1_torch.shape[2]
    assert k == 3 and w == 128, "kernel specialized to k=3, W=128 lanes"
    hw = h * w
    kkcin = k * k * cin
    lane_tile = min(16384, hw)
    assert hw % lane_tile == 0

    w1r = jnp.transpose(w1_torch, (0, 2, 3, 1)).reshape(ctmp, kkcin)
    w2m = w2_torch[:, :, 0, 0]
    b2c = b2.reshape(cout, 1).astype(jnp.float32)
    alpha_col = jnp.broadcast_to(alpha.astype(jnp.float32).reshape(1, 1),
                                 (cout, 1))
    wtail = jnp.concatenate([w2m, b2c, alpha_col], axis=1)
    wtail = jnp.pad(wtail, ((0, 0), (0, kkcin - ctmp - 2)))
    wpack = jnp.concatenate([w1r, wtail], axis=0)

    body = functools.partial(_fused_kernel, cin=cin, cout=cout, ctmp=ctmp,
                             h=h, w=w, lane_tile=lane_tile)
    out = pl.pallas_call(
        body,
        out_shape=jax.ShapeDtypeStruct((n, cout, h, w), x_nchw.dtype),
        grid_spec=pltpu.PrefetchScalarGridSpec(
            num_scalar_prefetch=0,
            grid=(n,),
            in_specs=[
                pl.BlockSpec((1, cin, h, w), lambda i: (i, 0, 0, 0)),
                pl.BlockSpec((ctmp + cout, kkcin), lambda i: (0, 0)),
            ],
            out_specs=pl.BlockSpec((1, cout, h, w), lambda i: (i, 0, 0, 0)),
        ),
        compiler_params=pltpu.CompilerParams(
            dimension_semantics=("parallel",)),
    )(x_nchw, wpack)
    return out


# speedup vs baseline: 1.0232x; 1.0232x over previous
import functools

import jax
import jax.numpy as jnp
from jax.experimental import pallas as pl
from jax.experimental.pallas import tpu as pltpu


def _fused_kernel(x_ref, wp_ref, o_ref, *, cin, cout, ctmp, h, w, lane_tile):
    hw = h * w
    w1 = wp_ref[0:ctmp, :].astype(jnp.bfloat16)
    w2 = wp_ref[ctmp:ctmp + cout, 0:ctmp].astype(jnp.bfloat16)
    b2 = wp_ref[ctmp:ctmp + cout, ctmp:ctmp + 1].astype(jnp.bfloat16)
    alpha = wp_ref[ctmp:ctmp + 1, ctmp + 1:ctmp + 2][0, 0].astype(jnp.bfloat16)
    wc = jnp.dot(w2, w1, preferred_element_type=jnp.float32)
    wc = wc.astype(jnp.bfloat16)

    xb = x_ref[0].astype(jnp.bfloat16).reshape(cin, hw)
    zrow = jnp.zeros((cin, w), jnp.bfloat16)
    xp = jnp.concatenate([zrow, xb, zrow], axis=1)
    ln = hw + 2 * w

    col = jax.lax.broadcasted_iota(jnp.int32, (cin, ln), 1) & (w - 1)
    z1 = jnp.zeros((cin, 1), jnp.bfloat16)
    zb = jnp.bfloat16(0)
    d0 = jnp.concatenate([z1, xp[:, :-1]], axis=1)
    d0 = jnp.where(col == 0, zb, d0)
    d2 = jnp.concatenate([xp[:, 1:], z1], axis=1)
    d2 = jnp.where(col == w - 1, zb, d2)
    dstack = jnp.concatenate([d0, xp, d2], axis=0)

    for p0 in range(0, hw, lane_tile):
        y = None
        for dy in range(3):
            rhs = dstack[:, p0 + dy * w: p0 + dy * w + lane_tile]
            part = jnp.dot(wc[:, dy * 3 * cin:(dy + 1) * 3 * cin], rhs,
                           preferred_element_type=jnp.float32)
            y = part if y is None else y + part
        y = y.astype(jnp.bfloat16) + b2
        y = jnp.where(y >= 0, y, alpha * y)
        y = y.reshape(cout, lane_tile // w, w).astype(o_ref.dtype)
        o_ref[0, :, p0 // w:(p0 + lane_tile) // w, :] = y


def kernel(x_nchw, w1_torch, w2_torch, b2, alpha):
    n, cin, h, w = x_nchw.shape
    ctmp = w1_torch.shape[0]
    cout = w2_torch.shape[0]
    k = w1_torch.shape[2]
    assert k == 3 and w == 128, "kernel specialized to k=3, W=128 lanes"
    hw = h * w
    kkcin = k * k * cin
    lane_tile = min(16384, hw)
    assert hw % lane_tile == 0

    w1r = jnp.transpose(w1_torch, (0, 2, 3, 1)).reshape(ctmp, kkcin)
    w2m = w2_torch[:, :, 0, 0]
    b2c = b2.reshape(cout, 1).astype(jnp.float32)
    alpha_col = jnp.broadcast_to(alpha.astype(jnp.float32).reshape(1, 1),
                                 (cout, 1))
    wtail = jnp.concatenate([w2m, b2c, alpha_col], axis=1)
    wtail = jnp.pad(wtail, ((0, 0), (0, kkcin - ctmp - 2)))
    wpack = jnp.concatenate([w1r, wtail], axis=0)

    body = functools.partial(_fused_kernel, cin=cin, cout=cout, ctmp=ctmp,
                             h=h, w=w, lane_tile=lane_tile)
    out = pl.pallas_call(
        body,
        out_shape=jax.ShapeDtypeStruct((n, cout, h, w), x_nchw.dtype),
        grid_spec=pltpu.PrefetchScalarGridSpec(
            num_scalar_prefetch=0,
            grid=(n,),
            in_specs=[
                pl.BlockSpec((1, cin, h, w), lambda i: (i, 0, 0, 0)),
                pl.BlockSpec((ctmp + cout, kkcin), lambda i: (0, 0)),
            ],
            out_specs=pl.BlockSpec((1, cout, h, w), lambda i: (i, 0, 0, 0)),
        ),
        compiler_params=pltpu.CompilerParams(
            dimension_semantics=("parallel",),
            allow_input_fusion=[True, True]),
    )(x_nchw, wpack)
    return out
